# baseline (speedup 1.0000x reference)
import functools

import jax
import jax.numpy as jnp
from jax import lax
from jax.experimental import pallas as pl
from jax.experimental.pallas import tpu as pltpu
from jax.experimental.pallas import tpu_sc as plsc

_B = 1024
_D = 64
_N_NEG = 5
_TOTAL = _B * (2 + _N_NEG)

_NC = 2
_NS = 16
_NW = _NC * _NS
_RPW = _TOTAL // _NW
_BAT = 112
_NB = _RPW // _BAT

_TN = 512
_NT = (_TOTAL - _B) // _TN
_POS_T = _B // _TN


_HPC = _TOTAL // 2
_CHK = 512


def _sc_gather(table, idx):
    mesh = plsc.ScalarSubcoreMesh(axis_name="c")

    @functools.partial(
        pl.kernel,
        out_type=jax.ShapeDtypeStruct((_TOTAL, _D), jnp.float32),
        mesh=mesh,
        scratch_types=[
            pltpu.SMEM((_CHK,), jnp.int32),
            pltpu.VMEM_SHARED((_HPC, _D), jnp.float32),
            pltpu.SemaphoreType.DMA,
        ],
    )
    def gather_k(table_hbm, idx_hbm, out_hbm, idx_s, stage, gsem):
        cid = lax.axis_index("c")
        base = cid * _HPC
        for ch in range(_HPC // _CHK):
            pltpu.sync_copy(
                idx_hbm.at[pl.ds(base + ch * _CHK, _CHK)], idx_s
            )

            def issue(r, carry, _ch=ch):
                i = idx_s[r]
                pltpu.async_copy(
                    table_hbm.at[pl.ds(i, 1)],
                    stage.at[pl.ds(_ch * _CHK + r, 1)],
                    gsem,
                )
                return carry

            lax.fori_loop(0, _CHK, issue, 0)

            def drain(r, carry):
                pltpu.make_async_copy(
                    table_hbm.at[pl.ds(0, 1)],
                    stage.at[pl.ds(0, 1)],
                    gsem,
                ).wait()
                return carry

            lax.fori_loop(0, _CHK, drain, 0)
        pltpu.sync_copy(stage, out_hbm.at[pl.ds(base, _HPC)])

    return gather_k(table, idx)


def _tc_body(lhs_ref, rhs_ref, out_ref):
    i = pl.program_id(0)
    logits = lax.dot_general(
        lhs_ref[...], rhs_ref[...],
        (((1,), (1,)), ((), ())),
        preferred_element_type=jnp.float32,
    )
    is_pos = i < _POS_T
    sign = jnp.where(is_pos, -1.0, 1.0).astype(jnp.float32)
    x = logits * sign
    tile_sum = jnp.sum(jnp.maximum(x, 0.0) + jnp.log1p(jnp.exp(-jnp.abs(x))))
    w = jnp.where(
        is_pos, 0.5 / (_B * _B), 0.5 / (_B * _B * _N_NEG)
    ).astype(jnp.float32)

    @pl.when(i == 0)
    def _():
        out_ref[...] = jnp.zeros_like(out_ref)

    out_ref[...] += jnp.full((1, 1), tile_sum * w, jnp.float32)


def _tc_loss(rows):
    return pl.pallas_call(
        _tc_body,
        grid=(_NT,),
        in_specs=[
            pl.BlockSpec((_B, _D), lambda i: (0, 0)),
            pl.BlockSpec((_TN, _D), lambda i: (i + _POS_T, 0)),
        ],
        out_specs=pl.BlockSpec((1, 1), lambda i: (0, 0)),
        out_shape=jax.ShapeDtypeStruct((1, 1), jnp.float32),
    )(rows, rows)


def kernel(src, pos, neg, table):
    idx = jnp.concatenate([src, pos, neg.reshape(-1)])
    rows = _sc_gather(table, idx)
    return _tc_loss(rows)[0, 0]
